# baseline (speedup 1.0000x reference)
import jax
import jax.numpy as jnp
from jax import lax
from jax.experimental import pallas as pl
from jax.experimental.pallas import tpu as pltpu
from jax.experimental.pallas import tpu_sc as plsc

_D = 4096
_LOW = 1024
_HIGH = _D - _LOW
_OUT = 3 * _HIGH + _LOW
_NC = 2
_NS = 16
_NW = _NC * _NS
_RC = 4
_L = 16


def _sc_body(src_hbm, thrv_hbm, emb_hbm, out_hbm, src_v, thrv_v, x_v, y_v):
    b = emb_hbm.shape[0] // _D
    rpw = b // _NW
    wid = lax.axis_index("s") * _NC + lax.axis_index("c")
    pltpu.sync_copy(src_hbm, src_v)
    pltpu.sync_copy(thrv_hbm, thrv_v)
    base = wid * rpw

    def chunk(ci, carry):
        row0 = base + ci * _RC
        pltpu.sync_copy(emb_hbm.at[pl.ds(row0 * _D, _RC * _D)], x_v)

        def group(g, carry2):
            off = g * _L
            idx = src_v[pl.ds(off, _L)]
            t = thrv_v[pl.ds(off, _L)]
            for rr in range(_RC):
                xg = plsc.load_gather(x_v, [idx + jnp.int32(rr * _D)])
                y_v[pl.ds(rr * _OUT + off, _L)] = (xg > t).astype(jnp.float32)
            return carry2

        lax.fori_loop(0, _OUT // _L, group, 0, unroll=4)
        pltpu.sync_copy(y_v, out_hbm.at[pl.ds(row0 * _OUT, _RC * _OUT)])
        return carry

    lax.fori_loop(0, rpw // _RC, chunk, 0)


def kernel(embeddings, thresholds, high_info_dims, low_info_dims):
    B = embeddings.shape[0]
    src_high = jnp.repeat(high_info_dims.astype(jnp.int32), 3)
    src = jnp.concatenate([src_high, low_info_dims.astype(jnp.int32)])
    thr_high = jnp.flip(jnp.take(thresholds, high_info_dims, axis=0), 1)
    thr_low = jnp.take(thresholds[:, 1], low_info_dims, axis=0)
    thrv = jnp.concatenate([thr_high.reshape(-1), thr_low])

    mesh = plsc.VectorSubcoreMesh(
        core_axis_name="c", subcore_axis_name="s",
        num_cores=_NC, num_subcores=_NS)
    run = pl.kernel(
        _sc_body,
        out_type=jax.ShapeDtypeStruct((B * _OUT,), jnp.float32),
        mesh=mesh,
        compiler_params=pltpu.CompilerParams(needs_layout_passes=False),
        scratch_types=[
            pltpu.VMEM((_OUT,), jnp.int32),
            pltpu.VMEM((_OUT,), jnp.float32),
            pltpu.VMEM((_RC * _D,), jnp.float32),
            pltpu.VMEM((_RC * _OUT,), jnp.float32),
        ],
    )
    out_flat = run(src, thrv, embeddings.reshape(-1))
    return out_flat.reshape(B, _OUT)


# speedup vs baseline: 2.2698x; 2.2698x over previous
import jax
import jax.numpy as jnp
from jax import lax
from jax.experimental import pallas as pl
from jax.experimental.pallas import tpu as pltpu
from jax.experimental.pallas import tpu_sc as plsc

_D = 4096
_LOW = 1024
_HIGH = _D - _LOW
_OUT = 3 * _HIGH + _LOW
_NC = 2
_NS = 16
_NW = _NC * _NS
_RC = 4
_L = 16


def _sc_body(src_hbm, thrv_hbm, emb_hbm, out_hbm,
             src_v, thrv_v, xb0, xb1, y_v, sx0, sx1, sy):
    b = emb_hbm.shape[0] // _D
    rpw = b // _NW
    nch = rpw // _RC
    wid = lax.axis_index("s") * _NC + lax.axis_index("c")
    base = wid * rpw
    pltpu.sync_copy(src_hbm, src_v)
    pltpu.sync_copy(thrv_hbm, thrv_v)

    def start_x(ci, xb, sem):
        row0 = base + ci * _RC
        pltpu.async_copy(emb_hbm.at[pl.ds(row0 * _D, _RC * _D)], xb, sem)

    def wait_x(xb, sem):
        pltpu.make_async_copy(
            emb_hbm.at[pl.ds(0, _RC * _D)], xb, sem).wait()

    def start_y(ci):
        row0 = base + ci * _RC
        pltpu.async_copy(
            y_v, out_hbm.at[pl.ds(row0 * _OUT, _RC * _OUT)], sy)

    def wait_y():
        pltpu.make_async_copy(
            y_v, out_hbm.at[pl.ds(0, _RC * _OUT)], sy).wait()

    def compute(xb):
        @plsc.parallel_loop(0, _OUT // _L, unroll=8)
        def _(g):
            off = g * _L
            idx = src_v[pl.ds(off, _L)]
            t = thrv_v[pl.ds(off, _L)]
            for rr in range(_RC):
                xg = plsc.load_gather(xb, [idx + jnp.int32(rr * _D)])
                y_v[pl.ds(rr * _OUT + off, _L)] = (xg > t).astype(jnp.float32)

    start_x(0, xb0, sx0)
    last = nch - 1

    def pair(pi, carry):
        ci0 = pi * 2
        wait_x(xb0, sx0)
        start_x(jnp.minimum(ci0 + 1, last), xb1, sx1)
        pl.when(ci0 > 0)(wait_y)
        compute(xb0)
        start_y(ci0)
        wait_x(xb1, sx1)
        start_x(jnp.minimum(ci0 + 2, last), xb0, sx0)
        wait_y()
        compute(xb1)
        start_y(ci0 + 1)
        return carry

    lax.fori_loop(0, nch // 2, pair, 0)
    wait_y()
    wait_x(xb0, sx0)


def kernel(embeddings, thresholds, high_info_dims, low_info_dims):
    B = embeddings.shape[0]
    src_high = jnp.repeat(high_info_dims.astype(jnp.int32), 3)
    src = jnp.concatenate([src_high, low_info_dims.astype(jnp.int32)])
    thr_high = jnp.flip(jnp.take(thresholds, high_info_dims, axis=0), 1)
    thr_low = jnp.take(thresholds[:, 1], low_info_dims, axis=0)
    thrv = jnp.concatenate([thr_high.reshape(-1), thr_low])

    mesh = plsc.VectorSubcoreMesh(
        core_axis_name="c", subcore_axis_name="s",
        num_cores=_NC, num_subcores=_NS)
    run = pl.kernel(
        _sc_body,
        out_type=jax.ShapeDtypeStruct((B * _OUT,), jnp.float32),
        mesh=mesh,
        compiler_params=pltpu.CompilerParams(needs_layout_passes=False),
        scratch_types=[
            pltpu.VMEM((_OUT,), jnp.int32),
            pltpu.VMEM((_OUT,), jnp.float32),
            pltpu.VMEM((_RC * _D,), jnp.float32),
            pltpu.VMEM((_RC * _D,), jnp.float32),
            pltpu.VMEM((_RC * _OUT,), jnp.float32),
            pltpu.SemaphoreType.DMA,
            pltpu.SemaphoreType.DMA,
            pltpu.SemaphoreType.DMA,
        ],
    )
    out_flat = run(src, thrv, embeddings.reshape(-1))
    return out_flat.reshape(B, _OUT)
